# Initial kernel scaffold; baseline (speedup 1.0000x reference)
import jax
import jax.numpy as jnp
from jax.experimental import pallas as pl


def kernel(x, edge_index, W, att_src, att_dst, bias, ln_gamma, ln_beta):
    raise NotImplementedError("write your pallas kernel here")


# speedup vs baseline: 10.9015x; 10.9015x over previous
import functools

import jax
import jax.numpy as jnp
from jax import lax
from jax.experimental import pallas as pl
from jax.experimental.pallas import tpu as pltpu
from jax.experimental.pallas import tpu_sc as plsc

N = 10000
E = 160000
IN_DIM = 256
HEADS = 8
OUT_DIM = 64
HD = HEADS * OUT_DIM
NEG_SLOPE = 0.2
LN_EPS = 1e-5

N_PAD = 10240
NTILES = 32
ROWS_PT = N_PAD // 16
CB = 128
EPT = 5376
NCH = EPT // CB
E_PAD = NTILES * EPT

RA = 1280


def _stage_a_body(x_ref, w_ref, asrc_w_ref, adst_w_ref, h_ref, asrc_ref, adst_ref):
    h = jnp.dot(x_ref[...], w_ref[...], preferred_element_type=jnp.float32)
    h_ref[...] = h
    h3 = h.reshape(RA, HEADS, OUT_DIM)
    a_s = jnp.sum(h3 * asrc_w_ref[...][None, :, :], axis=-1)
    a_d = jnp.sum(h3 * adst_w_ref[...][None, :, :], axis=-1)
    asrc_ref[...] = jnp.concatenate([a_s, a_s], axis=1)
    adst_ref[...] = jnp.concatenate([a_d, a_d], axis=1)


_stage_a = pl.pallas_call(
    _stage_a_body,
    grid=(N_PAD // RA,),
    in_specs=[
        pl.BlockSpec((RA, IN_DIM), lambda i: (i, 0)),
        pl.BlockSpec((IN_DIM, HD), lambda i: (0, 0)),
        pl.BlockSpec((HEADS, OUT_DIM), lambda i: (0, 0)),
        pl.BlockSpec((HEADS, OUT_DIM), lambda i: (0, 0)),
    ],
    out_specs=[
        pl.BlockSpec((RA, HD), lambda i: (i, 0)),
        pl.BlockSpec((RA, 16), lambda i: (i, 0)),
        pl.BlockSpec((RA, 16), lambda i: (i, 0)),
    ],
    out_shape=[
        jax.ShapeDtypeStruct((N_PAD, HD), jnp.float32),
        jax.ShapeDtypeStruct((N_PAD, 16), jnp.float32),
        jax.ShapeDtypeStruct((N_PAD, 16), jnp.float32),
    ],
)


def _sc_body(src_hbm, dst_hbm, asrc_hbm, adst_hbm, h_hbm,
             w_hbm, den_out, agg_out,
             srcv, dstv, gidx, asr, adr, wbuf, hrows,
             zb16, zb64, den_sh, agg_sh, sem):
    cid = lax.axis_index("c")
    sid = lax.axis_index("s")
    tid = cid * 16 + sid
    row0 = sid * ROWS_PT

    z16 = jnp.zeros((16,), jnp.float32)

    def zfill(r, carry):
        zb16[r, :] = z16
        for k in range(4):
            zb64[r, pl.ds(k * 16, 16)] = z16
        return carry

    lax.fori_loop(0, ROWS_PT, zfill, 0)

    pltpu.sync_copy(zb16, den_sh.at[pl.ds(row0, ROWS_PT)])
    plsc.subcore_barrier()

    def pass1_chunk(i, carry):
        base = (tid * NCH + i) * CB
        pltpu.sync_copy(src_hbm.at[pl.ds(base, CB)], srcv)
        pltpu.sync_copy(dst_hbm.at[pl.ds(base, CB)], dstv)
        pltpu.async_copy(asrc_hbm.at[srcv], asr, sem).wait()
        pltpu.async_copy(adst_hbm.at[dstv], adr, sem).wait()

        def edge(e, c2):
            s = asr[e, :] + adr[e, :]
            s = jnp.where(s >= 0.0, s, s * NEG_SLOPE)
            wbuf[e, :] = jnp.exp(s)
            return c2

        lax.fori_loop(0, CB, edge, 0)
        pltpu.sync_copy(wbuf, w_hbm.at[pl.ds((tid * NCH + i) * CB, CB)])
        pltpu.sync_copy(wbuf, den_sh.at[dstv], add=True)
        return carry

    lax.fori_loop(0, NCH, pass1_chunk, 0)
    plsc.subcore_barrier()
    pltpu.sync_copy(den_sh.at[pl.ds(row0, ROWS_PT)],
                    den_out.at[cid, pl.ds(row0, ROWS_PT)])

    for h in range(HEADS):
        pltpu.sync_copy(zb64, agg_sh.at[pl.ds(row0, ROWS_PT)])
        plsc.subcore_barrier()

        def chunk(i, carry, h=h):
            base = (tid * NCH + i) * CB
            pltpu.sync_copy(src_hbm.at[pl.ds(base, CB)], srcv)
            pltpu.sync_copy(dst_hbm.at[pl.ds(base, CB)], dstv)
            pltpu.sync_copy(w_hbm.at[pl.ds(base, CB)], wbuf)
            for k in range(CB // 16):
                gidx[pl.ds(k * 16, 16)] = srcv[pl.ds(k * 16, 16)] * HEADS + h
            pltpu.async_copy(h_hbm.at[gidx], hrows, sem).wait()

            def scale(e, c2, h=h):
                w = wbuf[e, :][h]
                for k in range(4):
                    hrows[e, pl.ds(k * 16, 16)] = hrows[e, pl.ds(k * 16, 16)] * w
                return c2

            lax.fori_loop(0, CB, scale, 0)
            pltpu.sync_copy(hrows, agg_sh.at[dstv], add=True)
            return carry

        lax.fori_loop(0, NCH, chunk, 0)
        plsc.subcore_barrier()
        pltpu.sync_copy(agg_sh.at[pl.ds(row0, ROWS_PT)],
                        agg_out.at[cid, h, pl.ds(row0, ROWS_PT)])
        plsc.subcore_barrier()


_stage_b = pl.kernel(
    _sc_body,
    out_type=(
        jax.ShapeDtypeStruct((E_PAD, 16), jnp.float32),
        jax.ShapeDtypeStruct((2, N_PAD, 16), jnp.float32),
        jax.ShapeDtypeStruct((2, HEADS, N_PAD, OUT_DIM), jnp.float32),
    ),
    mesh=plsc.VectorSubcoreMesh(core_axis_name="c", subcore_axis_name="s"),
    compiler_params=pltpu.CompilerParams(use_tc_tiling_on_sc=False),
    scratch_types=[
        pltpu.VMEM((CB,), jnp.int32),
        pltpu.VMEM((CB,), jnp.int32),
        pltpu.VMEM((CB,), jnp.int32),
        pltpu.VMEM((CB, 16), jnp.float32),
        pltpu.VMEM((CB, 16), jnp.float32),
        pltpu.VMEM((CB, 16), jnp.float32),
        pltpu.VMEM((CB, OUT_DIM), jnp.float32),
        pltpu.VMEM((ROWS_PT, 16), jnp.float32),
        pltpu.VMEM((ROWS_PT, 64), jnp.float32),
        pltpu.VMEM_SHARED((N_PAD, 16), jnp.float32),
        pltpu.VMEM_SHARED((N_PAD, 64), jnp.float32),
        pltpu.SemaphoreType.DMA,
    ],
)


def _stage_c_body(agg_ref, den_ref, bias_ref, gamma_ref, beta_ref, out_ref):
    s = agg_ref[0] + agg_ref[1]
    den16 = den_ref[0] + den_ref[1]
    eye = jnp.eye(HEADS, 16, dtype=jnp.float32)
    dent = lax.dot_general(eye, den16, (((1,), (1,)), ((), ())),
                           preferred_element_type=jnp.float32)
    y = s / dent[:, :, None] + bias_ref[...][:, None, :]
    t = jnp.sum(y, axis=0)
    mu = jnp.sum(t, axis=1) * (1.0 / HD)
    yc = y - mu[None, :, None]
    t2 = jnp.sum(yc * yc, axis=0)
    var = jnp.sum(t2, axis=1) * (1.0 / HD)
    inv = lax.rsqrt(var + LN_EPS)
    out_ref[...] = (yc * inv[None, :, None] * gamma_ref[...][:, None, :]
                    + beta_ref[...][:, None, :])


_stage_c = pl.pallas_call(
    _stage_c_body,
    grid=(N_PAD // RA,),
    in_specs=[
        pl.BlockSpec((2, HEADS, RA, OUT_DIM), lambda i: (0, 0, i, 0)),
        pl.BlockSpec((2, RA, 16), lambda i: (0, i, 0)),
        pl.BlockSpec((HEADS, OUT_DIM), lambda i: (0, 0)),
        pl.BlockSpec((HEADS, OUT_DIM), lambda i: (0, 0)),
        pl.BlockSpec((HEADS, OUT_DIM), lambda i: (0, 0)),
    ],
    out_specs=pl.BlockSpec((HEADS, RA, OUT_DIM), lambda i: (0, i, 0)),
    out_shape=jax.ShapeDtypeStruct((HEADS, N_PAD, OUT_DIM), jnp.float32),
)


def kernel(x, edge_index, W, att_src, att_dst, bias, ln_gamma, ln_beta):
    x_pad = jnp.zeros((N_PAD, IN_DIM), jnp.float32).at[:N].set(x)
    h, asrc, adst = _stage_a(x_pad, W, att_src, att_dst)

    src = edge_index[0].astype(jnp.int32)
    dst = edge_index[1].astype(jnp.int32)
    loop = jnp.arange(N, dtype=jnp.int32)
    pad = jnp.full((E_PAD - E - N,), N, dtype=jnp.int32)
    src_full = jnp.concatenate([src, loop, pad])
    dst_full = jnp.concatenate([dst, loop, pad])

    h2d = h.reshape(N_PAD * HEADS, OUT_DIM)
    _, den_p, agg_p = _stage_b(src_full, dst_full, asrc, adst, h2d)

    out8 = _stage_c(agg_p, den_p,
                    bias.reshape(HEADS, OUT_DIM),
                    ln_gamma.reshape(HEADS, OUT_DIM),
                    ln_beta.reshape(HEADS, OUT_DIM))
    return out8[:, :N, :].transpose(1, 0, 2).reshape(N, HD)


# speedup vs baseline: 14.3144x; 1.3131x over previous
import functools

import jax
import jax.numpy as jnp
from jax import lax
from jax.experimental import pallas as pl
from jax.experimental.pallas import tpu as pltpu
from jax.experimental.pallas import tpu_sc as plsc

N = 10000
E = 160000
IN_DIM = 256
HEADS = 8
OUT_DIM = 64
HD = HEADS * OUT_DIM
NEG_SLOPE = 0.2
LN_EPS = 1e-5

N_PAD = 10240
NTILES = 32
ROWS_PT = N_PAD // 16
CB = 128
EPT = 5376
NCH = EPT // CB
E_PAD = NTILES * EPT

RA = 1280
GROUPS = 8
ZROWS = 128


def _stage_a_body(x_ref, w_ref, asrc_w_ref, adst_w_ref, h_ref, asrc_ref, adst_ref):
    h = jnp.dot(x_ref[...], w_ref[...], preferred_element_type=jnp.float32)
    h_ref[...] = h
    h3 = h.reshape(RA, HEADS, OUT_DIM)
    a_s = jnp.sum(h3 * asrc_w_ref[...][None, :, :], axis=-1)
    a_d = jnp.sum(h3 * adst_w_ref[...][None, :, :], axis=-1)
    asrc_ref[...] = jnp.concatenate([a_s, a_s], axis=1)
    adst_ref[...] = jnp.concatenate([a_d, a_d], axis=1)


_stage_a = pl.pallas_call(
    _stage_a_body,
    grid=(N_PAD // RA,),
    in_specs=[
        pl.BlockSpec((RA, IN_DIM), lambda i: (i, 0)),
        pl.BlockSpec((IN_DIM, HD), lambda i: (0, 0)),
        pl.BlockSpec((HEADS, OUT_DIM), lambda i: (0, 0)),
        pl.BlockSpec((HEADS, OUT_DIM), lambda i: (0, 0)),
    ],
    out_specs=[
        pl.BlockSpec((RA, HD), lambda i: (i, 0)),
        pl.BlockSpec((RA, 16), lambda i: (i, 0)),
        pl.BlockSpec((RA, 16), lambda i: (i, 0)),
    ],
    out_shape=[
        jax.ShapeDtypeStruct((N_PAD, HD), jnp.float32),
        jax.ShapeDtypeStruct((N_PAD, 16), jnp.float32),
        jax.ShapeDtypeStruct((N_PAD, 16), jnp.float32),
    ],
)


def _sc_body(src_hbm, dst_hbm, asrc_hbm, adst_hbm, h_hbm,
             w_hbm, den_out, agg_out,
             srcA, dstA, gidxA, srcB, dstB, gidxB,
             asr, adr, wA, wB, hrA, hrB, zb16, zb,
             den_sh, agg_sh,
             semLA, semLB, semGA, semGB, semSA, semSB):
    cid = lax.axis_index("c")
    sid = lax.axis_index("s")
    tid = cid * 16 + sid
    row0 = sid * ROWS_PT

    z16 = jnp.zeros((16,), jnp.float32)

    def zfill(r, carry):
        zb16[r, :] = z16
        return carry

    lax.fori_loop(0, ROWS_PT, zfill, 0)

    def zfill2(r, carry):
        for k in range(4):
            zb[r, pl.ds(k * 16, 16)] = z16
        return carry

    lax.fori_loop(0, ZROWS, zfill2, 0)

    pltpu.sync_copy(zb16, den_sh.at[pl.ds(row0, ROWS_PT)])
    plsc.subcore_barrier()

    def pass1_chunk(i, carry):
        base = (tid * NCH + i) * CB
        pltpu.sync_copy(src_hbm.at[pl.ds(base, CB)], srcA)
        pltpu.sync_copy(dst_hbm.at[pl.ds(base, CB)], dstA)
        pltpu.async_copy(asrc_hbm.at[srcA], asr, semLA).wait()
        pltpu.async_copy(adst_hbm.at[dstA], adr, semLA).wait()

        def edge(e, c2):
            s = asr[e, :] + adr[e, :]
            s = jnp.where(s >= 0.0, s, s * NEG_SLOPE)
            wA[e, :] = jnp.exp(s)
            return c2

        lax.fori_loop(0, CB, edge, 0, unroll=2)
        pltpu.sync_copy(wA, w_hbm.at[pl.ds(base, CB)])
        pltpu.sync_copy(wA, den_sh.at[dstA], add=True)
        return carry

    lax.fori_loop(0, NCH, pass1_chunk, 0)
    plsc.subcore_barrier()
    pltpu.sync_copy(den_sh.at[pl.ds(row0, ROWS_PT)],
                    den_out.at[cid, pl.ds(row0, ROWS_PT)])

    for g in range(GROUPS):
        for zi in range(ROWS_PT // ZROWS):
            pltpu.sync_copy(zb, agg_sh.at[pl.ds(row0 + zi * ZROWS, ZROWS)])
        plsc.subcore_barrier()

        def pair(j, carry, g=g):
            baseA = (tid * NCH + 2 * j) * CB
            baseB = baseA + CB
            dlA = [pltpu.async_copy(src_hbm.at[pl.ds(baseA, CB)], srcA, semLA),
                   pltpu.async_copy(dst_hbm.at[pl.ds(baseA, CB)], dstA, semLA),
                   pltpu.async_copy(w_hbm.at[pl.ds(baseA, CB)], wA, semLA)]
            dlB = [pltpu.async_copy(src_hbm.at[pl.ds(baseB, CB)], srcB, semLB),
                   pltpu.async_copy(dst_hbm.at[pl.ds(baseB, CB)], dstB, semLB),
                   pltpu.async_copy(w_hbm.at[pl.ds(baseB, CB)], wB, semLB)]
            for d in dlA:
                d.wait()
            for k in range(CB // 16):
                gidxA[pl.ds(k * 16, 16)] = srcA[pl.ds(k * 16, 16)] * GROUPS + g
            dgA = pltpu.async_copy(h_hbm.at[gidxA], hrA, semGA)
            for d in dlB:
                d.wait()
            for k in range(CB // 16):
                gidxB[pl.ds(k * 16, 16)] = srcB[pl.ds(k * 16, 16)] * GROUPS + g
            dgB = pltpu.async_copy(h_hbm.at[gidxB], hrB, semGB)

            dgA.wait()

            def scaleA(e, c2, g=g):
                w0 = wA[e, :][g]
                for k in range(4):
                    hrA[e, pl.ds(k * 16, 16)] = hrA[e, pl.ds(k * 16, 16)] * w0
                return c2

            lax.fori_loop(0, CB, scaleA, 0, unroll=2)
            dsA = pltpu.async_copy(hrA, agg_sh.at[dstA], semSA, add=True)

            dgB.wait()

            def scaleB(e, c2, g=g):
                w0 = wB[e, :][g]
                for k in range(4):
                    hrB[e, pl.ds(k * 16, 16)] = hrB[e, pl.ds(k * 16, 16)] * w0
                return c2

            lax.fori_loop(0, CB, scaleB, 0, unroll=2)
            dsB = pltpu.async_copy(hrB, agg_sh.at[dstB], semSB, add=True)
            dsA.wait()
            dsB.wait()
            return carry

        lax.fori_loop(0, NCH // 2, pair, 0)
        plsc.subcore_barrier()
        pltpu.sync_copy(agg_sh.at[pl.ds(row0, ROWS_PT)],
                        agg_out.at[cid, g, pl.ds(row0, ROWS_PT)])
        plsc.subcore_barrier()


_stage_b = pl.kernel(
    _sc_body,
    out_type=(
        jax.ShapeDtypeStruct((E_PAD, 16), jnp.float32),
        jax.ShapeDtypeStruct((2, N_PAD, 16), jnp.float32),
        jax.ShapeDtypeStruct((2, HEADS, N_PAD, OUT_DIM), jnp.float32),
    ),
    mesh=plsc.VectorSubcoreMesh(core_axis_name="c", subcore_axis_name="s"),
    compiler_params=pltpu.CompilerParams(use_tc_tiling_on_sc=False),
    scratch_types=[
        pltpu.VMEM((CB,), jnp.int32),
        pltpu.VMEM((CB,), jnp.int32),
        pltpu.VMEM((CB,), jnp.int32),
        pltpu.VMEM((CB,), jnp.int32),
        pltpu.VMEM((CB,), jnp.int32),
        pltpu.VMEM((CB,), jnp.int32),
        pltpu.VMEM((CB, 16), jnp.float32),
        pltpu.VMEM((CB, 16), jnp.float32),
        pltpu.VMEM((CB, 16), jnp.float32),
        pltpu.VMEM((CB, 16), jnp.float32),
        pltpu.VMEM((CB, OUT_DIM), jnp.float32),
        pltpu.VMEM((CB, OUT_DIM), jnp.float32),
        pltpu.VMEM((ROWS_PT, 16), jnp.float32),
        pltpu.VMEM((ZROWS, OUT_DIM), jnp.float32),
        pltpu.VMEM_SHARED((N_PAD, 16), jnp.float32),
        pltpu.VMEM_SHARED((N_PAD, OUT_DIM), jnp.float32),
        pltpu.SemaphoreType.DMA,
        pltpu.SemaphoreType.DMA,
        pltpu.SemaphoreType.DMA,
        pltpu.SemaphoreType.DMA,
        pltpu.SemaphoreType.DMA,
        pltpu.SemaphoreType.DMA,
    ],
)


def _stage_c_body(agg_ref, den_ref, bias_ref, gamma_ref, beta_ref, out_ref):
    s = agg_ref[0] + agg_ref[1]
    den16 = den_ref[0] + den_ref[1]
    eye = jnp.eye(HEADS, 16, dtype=jnp.float32)
    dent = lax.dot_general(eye, den16, (((1,), (1,)), ((), ())),
                           preferred_element_type=jnp.float32)
    y = s / dent[:, :, None] + bias_ref[...][:, None, :]
    t = jnp.sum(y, axis=0)
    mu = jnp.sum(t, axis=1) * (1.0 / HD)
    yc = y - mu[None, :, None]
    t2 = jnp.sum(yc * yc, axis=0)
    var = jnp.sum(t2, axis=1) * (1.0 / HD)
    inv = lax.rsqrt(var + LN_EPS)
    out_ref[...] = (yc * inv[None, :, None] * gamma_ref[...][:, None, :]
                    + beta_ref[...][:, None, :])


_stage_c = pl.pallas_call(
    _stage_c_body,
    grid=(N_PAD // RA,),
    in_specs=[
        pl.BlockSpec((2, HEADS, RA, OUT_DIM), lambda i: (0, 0, i, 0)),
        pl.BlockSpec((2, RA, 16), lambda i: (0, i, 0)),
        pl.BlockSpec((HEADS, OUT_DIM), lambda i: (0, 0)),
        pl.BlockSpec((HEADS, OUT_DIM), lambda i: (0, 0)),
        pl.BlockSpec((HEADS, OUT_DIM), lambda i: (0, 0)),
    ],
    out_specs=pl.BlockSpec((HEADS, RA, OUT_DIM), lambda i: (0, i, 0)),
    out_shape=jax.ShapeDtypeStruct((HEADS, N_PAD, OUT_DIM), jnp.float32),
)


def kernel(x, edge_index, W, att_src, att_dst, bias, ln_gamma, ln_beta):
    x_pad = jnp.zeros((N_PAD, IN_DIM), jnp.float32).at[:N].set(x)
    h, asrc, adst = _stage_a(x_pad, W, att_src, att_dst)

    src = edge_index[0].astype(jnp.int32)
    dst = edge_index[1].astype(jnp.int32)
    loop = jnp.arange(N, dtype=jnp.int32)
    pad = jnp.full((E_PAD - E - N,), N, dtype=jnp.int32)
    src_full = jnp.concatenate([src, loop, pad])
    dst_full = jnp.concatenate([dst, loop, pad])

    h2d = h.reshape(N_PAD * GROUPS, OUT_DIM)
    _, den_p, agg_p = _stage_b(src_full, dst_full, asrc, adst, h2d)

    out8 = _stage_c(agg_p, den_p,
                    bias.reshape(HEADS, OUT_DIM),
                    ln_gamma.reshape(HEADS, OUT_DIM),
                    ln_beta.reshape(HEADS, OUT_DIM))
    return out8[:, :N, :].transpose(1, 0, 2).reshape(N, HD)


# speedup vs baseline: 15.6806x; 1.0954x over previous
import functools

import jax
import jax.numpy as jnp
from jax import lax
from jax.experimental import pallas as pl
from jax.experimental.pallas import tpu as pltpu
from jax.experimental.pallas import tpu_sc as plsc

N = 10000
E = 160000
IN_DIM = 256
HEADS = 8
OUT_DIM = 64
HD = HEADS * OUT_DIM
NEG_SLOPE = 0.2
LN_EPS = 1e-5

N_PAD = 10240
NTILES = 32
ROWS_PT = N_PAD // 16
CB = 128
EPT = 5376
NCH = EPT // CB
NPAIR = NCH // 2
E_PAD = NTILES * EPT

RA = 1280
GROUPS = 8
ZROWS = 128


def _stage_a_body(x_ref, w_ref, asrc_w_ref, adst_w_ref, h_ref, asrc_ref, adst_ref):
    h = jnp.dot(x_ref[...], w_ref[...], preferred_element_type=jnp.float32)
    h_ref[...] = h
    h3 = h.reshape(RA, HEADS, OUT_DIM)
    a_s = jnp.sum(h3 * asrc_w_ref[...][None, :, :], axis=-1)
    a_d = jnp.sum(h3 * adst_w_ref[...][None, :, :], axis=-1)
    asrc_ref[...] = jnp.concatenate([a_s, a_s], axis=1)
    adst_ref[...] = jnp.concatenate([a_d, a_d], axis=1)


_stage_a = pl.pallas_call(
    _stage_a_body,
    grid=(N_PAD // RA,),
    in_specs=[
        pl.BlockSpec((RA, IN_DIM), lambda i: (i, 0)),
        pl.BlockSpec((IN_DIM, HD), lambda i: (0, 0)),
        pl.BlockSpec((HEADS, OUT_DIM), lambda i: (0, 0)),
        pl.BlockSpec((HEADS, OUT_DIM), lambda i: (0, 0)),
    ],
    out_specs=[
        pl.BlockSpec((RA, HD), lambda i: (i, 0)),
        pl.BlockSpec((RA, 16), lambda i: (i, 0)),
        pl.BlockSpec((RA, 16), lambda i: (i, 0)),
    ],
    out_shape=[
        jax.ShapeDtypeStruct((N_PAD, HD), jnp.float32),
        jax.ShapeDtypeStruct((N_PAD, 16), jnp.float32),
        jax.ShapeDtypeStruct((N_PAD, 16), jnp.float32),
    ],
)


def _sc_body(src_hbm, dst_hbm, asrc_hbm, adst_hbm, h_hbm,
             den_out, agg_out,
             srcF, dstF, dstC, gidxP, asr, adr, wbuf, wall, hrA, hrB, zb,
             den_sh, agg_sh,
             semL, semGA, semGB, semSA, semSB):
    cid = lax.axis_index("c")
    sid = lax.axis_index("s")
    tid = cid * 16 + sid
    row0 = sid * ROWS_PT

    z16 = jnp.zeros((16,), jnp.float32)
    m8 = lax.iota(jnp.int32, 16) < 8

    def zfill(r, carry):
        for k in range(4):
            zb[r, pl.ds(k * 16, 16)] = z16
        return carry

    lax.fori_loop(0, ZROWS, zfill, 0)

    for zi in range(ROWS_PT // ZROWS):
        pltpu.sync_copy(zb.at[pl.ds(0, ZROWS), pl.ds(0, 16)],
                        den_sh.at[pl.ds(row0 + zi * ZROWS, ZROWS)])
    plsc.subcore_barrier()

    def pass1_chunk(i, carry):
        base = tid * EPT + i * CB
        d1 = pltpu.async_copy(src_hbm.at[pl.ds(base, CB)], srcF.at[pl.ds(0, CB)], semL)
        d2 = pltpu.async_copy(dst_hbm.at[pl.ds(base, CB)], dstC.at[0], semL)
        d1.wait()
        d2.wait()
        pltpu.async_copy(asrc_hbm.at[srcF.at[pl.ds(0, CB)]], asr, semGA).wait()
        pltpu.async_copy(adst_hbm.at[dstC.at[0]], adr, semGB).wait()

        def edge2(f, c2):
            e0 = 2 * f
            s0 = asr[e0, :] + adr[e0, :]
            s0 = jnp.where(s0 >= 0.0, s0, s0 * NEG_SLOPE)
            w0 = jnp.exp(s0)
            wbuf[e0, :] = w0
            s1 = asr[e0 + 1, :] + adr[e0 + 1, :]
            s1 = jnp.where(s1 >= 0.0, s1, s1 * NEG_SLOPE)
            w1 = jnp.exp(s1)
            wbuf[e0 + 1, :] = w1
            wall[pl.ds((i * CB + e0) * 8, 16)] = jnp.where(m8, w0, w1)
            return c2

        lax.fori_loop(0, CB // 2, edge2, 0)
        pltpu.sync_copy(wbuf, den_sh.at[dstC.at[0]], add=True)
        return carry

    lax.fori_loop(0, NCH, pass1_chunk, 0)
    plsc.subcore_barrier()
    pltpu.sync_copy(den_sh.at[pl.ds(row0, ROWS_PT)],
                    den_out.at[cid, pl.ds(row0, ROWS_PT)])

    for g in range(GROUPS):
        for zi in range(ROWS_PT // ZROWS):
            pltpu.sync_copy(zb, agg_sh.at[pl.ds(row0 + zi * ZROWS, ZROWS)])
        plsc.subcore_barrier()

        dL1 = pltpu.async_copy(src_hbm.at[pl.ds(tid * EPT, 2 * CB)], srcF, semL)
        dL2 = pltpu.async_copy(dst_hbm.at[pl.ds(tid * EPT, 2 * CB)], dstF, semL)

        def pair(j, carry, g=g):
            dL1.wait()
            dL2.wait()
            for k in range(2 * CB // 16):
                gidxP[pl.ds(k * 16, 16)] = srcF[pl.ds(k * 16, 16)] * GROUPS + g
            for r in range(2):
                for k in range(CB // 16):
                    dstC[r, pl.ds(k * 16, 16)] = dstF[pl.ds(r * CB + k * 16, 16)]
            dgA = pltpu.async_copy(h_hbm.at[gidxP.at[pl.ds(0, CB)]], hrA, semGA)
            dgB = pltpu.async_copy(h_hbm.at[gidxP.at[pl.ds(CB, CB)]], hrB, semGB)

            @pl.when(j + 1 < NPAIR)
            def _():
                nbase = tid * EPT + (2 * j + 2) * CB
                pltpu.async_copy(src_hbm.at[pl.ds(nbase, 2 * CB)], srcF, semL)
                pltpu.async_copy(dst_hbm.at[pl.ds(nbase, 2 * CB)], dstF, semL)

            dgA.wait()

            def scaleA(e, c2, g=g):
                w0 = wall[pl.ds((2 * j * CB + e) * 8, 16)][g]
                for k in range(4):
                    hrA[e, pl.ds(k * 16, 16)] = hrA[e, pl.ds(k * 16, 16)] * w0
                return c2

            lax.fori_loop(0, CB, scaleA, 0, unroll=2)
            dsA = pltpu.async_copy(hrA, agg_sh.at[dstC.at[0]], semSA, add=True)

            dgB.wait()

            def scaleB(e, c2, g=g):
                w0 = wall[pl.ds(((2 * j + 1) * CB + e) * 8, 16)][g]
                for k in range(4):
                    hrB[e, pl.ds(k * 16, 16)] = hrB[e, pl.ds(k * 16, 16)] * w0
                return c2

            lax.fori_loop(0, CB, scaleB, 0, unroll=2)
            dsB = pltpu.async_copy(hrB, agg_sh.at[dstC.at[1]], semSB, add=True)
            dsA.wait()
            dsB.wait()
            return carry

        lax.fori_loop(0, NPAIR, pair, 0)
        plsc.subcore_barrier()
        pltpu.sync_copy(agg_sh.at[pl.ds(row0, ROWS_PT)],
                        agg_out.at[cid, g, pl.ds(row0, ROWS_PT)])
        plsc.subcore_barrier()


_stage_b = pl.kernel(
    _sc_body,
    out_type=(
        jax.ShapeDtypeStruct((2, N_PAD, 16), jnp.float32),
        jax.ShapeDtypeStruct((2, HEADS, N_PAD, OUT_DIM), jnp.float32),
    ),
    mesh=plsc.VectorSubcoreMesh(core_axis_name="c", subcore_axis_name="s"),
    compiler_params=pltpu.CompilerParams(use_tc_tiling_on_sc=False),
    scratch_types=[
        pltpu.VMEM((2 * CB,), jnp.int32),
        pltpu.VMEM((2 * CB,), jnp.int32),
        pltpu.VMEM((2, CB), jnp.int32),
        pltpu.VMEM((2 * CB,), jnp.int32),
        pltpu.VMEM((CB, 16), jnp.float32),
        pltpu.VMEM((CB, 16), jnp.float32),
        pltpu.VMEM((CB, 16), jnp.float32),
        pltpu.VMEM((EPT * 8,), jnp.float32),
        pltpu.VMEM((CB, OUT_DIM), jnp.float32),
        pltpu.VMEM((CB, OUT_DIM), jnp.float32),
        pltpu.VMEM((ZROWS, OUT_DIM), jnp.float32),
        pltpu.VMEM_SHARED((N_PAD, 16), jnp.float32),
        pltpu.VMEM_SHARED((N_PAD, OUT_DIM), jnp.float32),
        pltpu.SemaphoreType.DMA,
        pltpu.SemaphoreType.DMA,
        pltpu.SemaphoreType.DMA,
        pltpu.SemaphoreType.DMA,
        pltpu.SemaphoreType.DMA,
    ],
)


def _stage_c_body(agg_ref, den_ref, bias_ref, gamma_ref, beta_ref, out_ref):
    s = agg_ref[0] + agg_ref[1]
    den16 = den_ref[0] + den_ref[1]
    eye = jnp.eye(HEADS, 16, dtype=jnp.float32)
    dent = lax.dot_general(eye, den16, (((1,), (1,)), ((), ())),
                           preferred_element_type=jnp.float32)
    y = s / dent[:, :, None] + bias_ref[...][:, None, :]
    t = jnp.sum(y, axis=0)
    mu = jnp.sum(t, axis=1) * (1.0 / HD)
    yc = y - mu[None, :, None]
    t2 = jnp.sum(yc * yc, axis=0)
    var = jnp.sum(t2, axis=1) * (1.0 / HD)
    inv = lax.rsqrt(var + LN_EPS)
    out_ref[...] = (yc * inv[None, :, None] * gamma_ref[...][:, None, :]
                    + beta_ref[...][:, None, :])


_stage_c = pl.pallas_call(
    _stage_c_body,
    grid=(N_PAD // RA,),
    in_specs=[
        pl.BlockSpec((2, HEADS, RA, OUT_DIM), lambda i: (0, 0, i, 0)),
        pl.BlockSpec((2, RA, 16), lambda i: (0, i, 0)),
        pl.BlockSpec((HEADS, OUT_DIM), lambda i: (0, 0)),
        pl.BlockSpec((HEADS, OUT_DIM), lambda i: (0, 0)),
        pl.BlockSpec((HEADS, OUT_DIM), lambda i: (0, 0)),
    ],
    out_specs=pl.BlockSpec((HEADS, RA, OUT_DIM), lambda i: (0, i, 0)),
    out_shape=jax.ShapeDtypeStruct((HEADS, N_PAD, OUT_DIM), jnp.float32),
)


def kernel(x, edge_index, W, att_src, att_dst, bias, ln_gamma, ln_beta):
    x_pad = jnp.zeros((N_PAD, IN_DIM), jnp.float32).at[:N].set(x)
    h, asrc, adst = _stage_a(x_pad, W, att_src, att_dst)

    src = edge_index[0].astype(jnp.int32)
    dst = edge_index[1].astype(jnp.int32)
    loop = jnp.arange(N, dtype=jnp.int32)
    pad = jnp.full((E_PAD - E - N,), N, dtype=jnp.int32)
    src_full = jnp.concatenate([src, loop, pad])
    dst_full = jnp.concatenate([dst, loop, pad])

    h2d = h.reshape(N_PAD * GROUPS, OUT_DIM)
    den_p, agg_p = _stage_b(src_full, dst_full, asrc, adst, h2d)

    out8 = _stage_c(agg_p, den_p,
                    bias.reshape(HEADS, OUT_DIM),
                    ln_gamma.reshape(HEADS, OUT_DIM),
                    ln_beta.reshape(HEADS, OUT_DIM))
    return out8[:, :N, :].transpose(1, 0, 2).reshape(N, HD)


# speedup vs baseline: 16.3130x; 1.0403x over previous
import functools

import jax
import jax.numpy as jnp
from jax import lax
from jax.experimental import pallas as pl
from jax.experimental.pallas import tpu as pltpu
from jax.experimental.pallas import tpu_sc as plsc

N = 10000
E = 160000
IN_DIM = 256
HEADS = 8
OUT_DIM = 64
HD = HEADS * OUT_DIM
NEG_SLOPE = 0.2
LN_EPS = 1e-5

N_PAD = 10240
NTILES = 32
ROWS_PT = N_PAD // 16
CB = 128
EPT = 5376
NCH = EPT // CB
NPAIR = NCH // 2
E_PAD = NTILES * EPT

RA = 1280
GROUPS = 8
ZROWS = 128


def _stage_a_body(x_ref, w_ref, asrc_w_ref, adst_w_ref, h_ref, asrc_ref, adst_ref):
    h = jnp.dot(x_ref[...], w_ref[...], preferred_element_type=jnp.float32)
    h_ref[...] = h
    h3 = h.reshape(RA, HEADS, OUT_DIM)
    a_s = jnp.sum(h3 * asrc_w_ref[...][None, :, :], axis=-1)
    a_d = jnp.sum(h3 * adst_w_ref[...][None, :, :], axis=-1)
    asrc_ref[...] = jnp.concatenate([a_s, a_s], axis=1)
    adst_ref[...] = jnp.concatenate([a_d, a_d], axis=1)


_stage_a = pl.pallas_call(
    _stage_a_body,
    grid=(N_PAD // RA,),
    in_specs=[
        pl.BlockSpec((RA, IN_DIM), lambda i: (i, 0)),
        pl.BlockSpec((IN_DIM, HD), lambda i: (0, 0)),
        pl.BlockSpec((HEADS, OUT_DIM), lambda i: (0, 0)),
        pl.BlockSpec((HEADS, OUT_DIM), lambda i: (0, 0)),
    ],
    out_specs=[
        pl.BlockSpec((RA, HD), lambda i: (i, 0)),
        pl.BlockSpec((RA, 16), lambda i: (i, 0)),
        pl.BlockSpec((RA, 16), lambda i: (i, 0)),
    ],
    out_shape=[
        jax.ShapeDtypeStruct((N_PAD, HD), jnp.float32),
        jax.ShapeDtypeStruct((N_PAD, 16), jnp.float32),
        jax.ShapeDtypeStruct((N_PAD, 16), jnp.float32),
    ],
)


def _sc_body(src_hbm, dst_hbm, asrc_hbm, adst_hbm, h_hbm,
             den_out, agg_out,
             srcF, dstF, dstC, gidxP, asr, adr, asrB, adrB,
             wbuf, wall, hrA, hrB, zb,
             den_sh, agg_sh,
             semL, semGA, semGB, semSA, semSB):
    cid = lax.axis_index("c")
    sid = lax.axis_index("s")
    tid = cid * 16 + sid
    row0 = sid * ROWS_PT

    z16 = jnp.zeros((16,), jnp.float32)
    m8 = lax.iota(jnp.int32, 16) < 8

    def zfill(r, carry):
        for k in range(4):
            zb[r, pl.ds(k * 16, 16)] = z16
        return carry

    lax.fori_loop(0, ZROWS, zfill, 0)

    for zi in range(ROWS_PT // ZROWS):
        pltpu.sync_copy(zb.at[pl.ds(0, ZROWS), pl.ds(0, 16)],
                        den_sh.at[pl.ds(row0 + zi * ZROWS, ZROWS)])
    plsc.subcore_barrier()

    dp1 = pltpu.async_copy(src_hbm.at[pl.ds(tid * EPT, 2 * CB)], srcF, semL)
    dp2 = pltpu.async_copy(dst_hbm.at[pl.ds(tid * EPT, 2 * CB)], dstF, semL)

    def pass1_pair(j, carry):
        dp1.wait()
        dp2.wait()
        for q in range(2):
            for k in range(CB // 16):
                dstC[q, pl.ds(k * 16, 16)] = dstF[pl.ds(q * CB + k * 16, 16)]
        gA1 = pltpu.async_copy(asrc_hbm.at[srcF.at[pl.ds(0, CB)]], asr, semGA)
        gA2 = pltpu.async_copy(adst_hbm.at[dstC.at[0]], adr, semGB)
        gB1 = pltpu.async_copy(asrc_hbm.at[srcF.at[pl.ds(CB, CB)]], asrB, semSA)
        gB2 = pltpu.async_copy(adst_hbm.at[dstC.at[1]], adrB, semSB)
        gA1.wait()
        gA2.wait()

        def edge2(f, c2):
            e0 = 2 * f
            s0 = asr[e0, :] + adr[e0, :]
            s0 = jnp.where(s0 >= 0.0, s0, s0 * NEG_SLOPE)
            w0 = jnp.exp(s0)
            wbuf[e0, :] = w0
            s1 = asr[e0 + 1, :] + adr[e0 + 1, :]
            s1 = jnp.where(s1 >= 0.0, s1, s1 * NEG_SLOPE)
            w1 = jnp.exp(s1)
            wbuf[e0 + 1, :] = w1
            wall[pl.ds((2 * j * CB + e0) * 8, 16)] = jnp.where(m8, w0, w1)
            return c2

        lax.fori_loop(0, CB // 2, edge2, 0)
        pltpu.sync_copy(wbuf, den_sh.at[dstC.at[0]], add=True)
        gB1.wait()
        gB2.wait()

        @pl.when(j + 1 < NPAIR)
        def _():
            nb = tid * EPT + (2 * j + 2) * CB
            pltpu.async_copy(src_hbm.at[pl.ds(nb, 2 * CB)], srcF, semL)
            pltpu.async_copy(dst_hbm.at[pl.ds(nb, 2 * CB)], dstF, semL)

        def edge2b(f, c2):
            e0 = 2 * f
            s0 = asrB[e0, :] + adrB[e0, :]
            s0 = jnp.where(s0 >= 0.0, s0, s0 * NEG_SLOPE)
            w0 = jnp.exp(s0)
            wbuf[e0, :] = w0
            s1 = asrB[e0 + 1, :] + adrB[e0 + 1, :]
            s1 = jnp.where(s1 >= 0.0, s1, s1 * NEG_SLOPE)
            w1 = jnp.exp(s1)
            wbuf[e0 + 1, :] = w1
            wall[pl.ds(((2 * j + 1) * CB + e0) * 8, 16)] = jnp.where(m8, w0, w1)
            return c2

        lax.fori_loop(0, CB // 2, edge2b, 0)
        pltpu.sync_copy(wbuf, den_sh.at[dstC.at[1]], add=True)
        return carry

    lax.fori_loop(0, NPAIR, pass1_pair, 0)
    plsc.subcore_barrier()
    pltpu.sync_copy(den_sh.at[pl.ds(row0, ROWS_PT)],
                    den_out.at[cid, pl.ds(row0, ROWS_PT)])

    for g in range(GROUPS):
        for zi in range(ROWS_PT // ZROWS):
            pltpu.sync_copy(zb, agg_sh.at[pl.ds(row0 + zi * ZROWS, ZROWS)])
        plsc.subcore_barrier()

        dL1 = pltpu.async_copy(src_hbm.at[pl.ds(tid * EPT, 2 * CB)], srcF, semL)
        dL2 = pltpu.async_copy(dst_hbm.at[pl.ds(tid * EPT, 2 * CB)], dstF, semL)

        def pair(j, carry, g=g):
            dL1.wait()
            dL2.wait()
            for k in range(2 * CB // 16):
                gidxP[pl.ds(k * 16, 16)] = srcF[pl.ds(k * 16, 16)] * GROUPS + g
            for r in range(2):
                for k in range(CB // 16):
                    dstC[r, pl.ds(k * 16, 16)] = dstF[pl.ds(r * CB + k * 16, 16)]
            dgA = pltpu.async_copy(h_hbm.at[gidxP.at[pl.ds(0, CB)]], hrA, semGA)
            dgB = pltpu.async_copy(h_hbm.at[gidxP.at[pl.ds(CB, CB)]], hrB, semGB)

            @pl.when(j + 1 < NPAIR)
            def _():
                nbase = tid * EPT + (2 * j + 2) * CB
                pltpu.async_copy(src_hbm.at[pl.ds(nbase, 2 * CB)], srcF, semL)
                pltpu.async_copy(dst_hbm.at[pl.ds(nbase, 2 * CB)], dstF, semL)

            dgA.wait()

            def scaleA(e, c2, g=g):
                w0 = wall[pl.ds((2 * j * CB + e) * 8, 16)][g]
                for k in range(4):
                    hrA[e, pl.ds(k * 16, 16)] = hrA[e, pl.ds(k * 16, 16)] * w0
                return c2

            lax.fori_loop(0, CB, scaleA, 0, unroll=2)
            dsA = pltpu.async_copy(hrA, agg_sh.at[dstC.at[0]], semSA, add=True)

            dgB.wait()

            def scaleB(e, c2, g=g):
                w0 = wall[pl.ds(((2 * j + 1) * CB + e) * 8, 16)][g]
                for k in range(4):
                    hrB[e, pl.ds(k * 16, 16)] = hrB[e, pl.ds(k * 16, 16)] * w0
                return c2

            lax.fori_loop(0, CB, scaleB, 0, unroll=2)
            dsB = pltpu.async_copy(hrB, agg_sh.at[dstC.at[1]], semSB, add=True)
            dsA.wait()
            dsB.wait()
            return carry

        lax.fori_loop(0, NPAIR, pair, 0)
        plsc.subcore_barrier()
        pltpu.sync_copy(agg_sh.at[pl.ds(row0, ROWS_PT)],
                        agg_out.at[cid, g, pl.ds(row0, ROWS_PT)])
        plsc.subcore_barrier()


_stage_b = pl.kernel(
    _sc_body,
    out_type=(
        jax.ShapeDtypeStruct((2, N_PAD, 16), jnp.float32),
        jax.ShapeDtypeStruct((2, HEADS, N_PAD, OUT_DIM), jnp.float32),
    ),
    mesh=plsc.VectorSubcoreMesh(core_axis_name="c", subcore_axis_name="s"),
    compiler_params=pltpu.CompilerParams(use_tc_tiling_on_sc=False),
    scratch_types=[
        pltpu.VMEM((2 * CB,), jnp.int32),
        pltpu.VMEM((2 * CB,), jnp.int32),
        pltpu.VMEM((2, CB), jnp.int32),
        pltpu.VMEM((2 * CB,), jnp.int32),
        pltpu.VMEM((CB, 16), jnp.float32),
        pltpu.VMEM((CB, 16), jnp.float32),
        pltpu.VMEM((CB, 16), jnp.float32),
        pltpu.VMEM((CB, 16), jnp.float32),
        pltpu.VMEM((CB, 16), jnp.float32),
        pltpu.VMEM((EPT * 8,), jnp.float32),
        pltpu.VMEM((CB, OUT_DIM), jnp.float32),
        pltpu.VMEM((CB, OUT_DIM), jnp.float32),
        pltpu.VMEM((ZROWS, OUT_DIM), jnp.float32),
        pltpu.VMEM_SHARED((N_PAD, 16), jnp.float32),
        pltpu.VMEM_SHARED((N_PAD, OUT_DIM), jnp.float32),
        pltpu.SemaphoreType.DMA,
        pltpu.SemaphoreType.DMA,
        pltpu.SemaphoreType.DMA,
        pltpu.SemaphoreType.DMA,
        pltpu.SemaphoreType.DMA,
    ],
)


def _stage_c_body(agg_ref, den_ref, bias_ref, gamma_ref, beta_ref, out_ref):
    s = agg_ref[0] + agg_ref[1]
    den16 = den_ref[0] + den_ref[1]
    eye = jnp.eye(HEADS, 16, dtype=jnp.float32)
    dent = lax.dot_general(eye, den16, (((1,), (1,)), ((), ())),
                           preferred_element_type=jnp.float32)
    y = s / dent[:, :, None] + bias_ref[...][:, None, :]
    t = jnp.sum(y, axis=0)
    mu = jnp.sum(t, axis=1) * (1.0 / HD)
    yc = y - mu[None, :, None]
    t2 = jnp.sum(yc * yc, axis=0)
    var = jnp.sum(t2, axis=1) * (1.0 / HD)
    inv = lax.rsqrt(var + LN_EPS)
    out_ref[...] = (yc * inv[None, :, None] * gamma_ref[...][:, None, :]
                    + beta_ref[...][:, None, :])


_stage_c = pl.pallas_call(
    _stage_c_body,
    grid=(N_PAD // RA,),
    in_specs=[
        pl.BlockSpec((2, HEADS, RA, OUT_DIM), lambda i: (0, 0, i, 0)),
        pl.BlockSpec((2, RA, 16), lambda i: (0, i, 0)),
        pl.BlockSpec((HEADS, OUT_DIM), lambda i: (0, 0)),
        pl.BlockSpec((HEADS, OUT_DIM), lambda i: (0, 0)),
        pl.BlockSpec((HEADS, OUT_DIM), lambda i: (0, 0)),
    ],
    out_specs=pl.BlockSpec((HEADS, RA, OUT_DIM), lambda i: (0, i, 0)),
    out_shape=jax.ShapeDtypeStruct((HEADS, N_PAD, OUT_DIM), jnp.float32),
)


def kernel(x, edge_index, W, att_src, att_dst, bias, ln_gamma, ln_beta):
    x_pad = jnp.zeros((N_PAD, IN_DIM), jnp.float32).at[:N].set(x)
    h, asrc, adst = _stage_a(x_pad, W, att_src, att_dst)

    src = edge_index[0].astype(jnp.int32)
    dst = edge_index[1].astype(jnp.int32)
    loop = jnp.arange(N, dtype=jnp.int32)
    pad = jnp.full((E_PAD - E - N,), N, dtype=jnp.int32)
    src_full = jnp.concatenate([src, loop, pad])
    dst_full = jnp.concatenate([dst, loop, pad])

    h2d = h.reshape(N_PAD * GROUPS, OUT_DIM)
    den_p, agg_p = _stage_b(src_full, dst_full, asrc, adst, h2d)

    out8 = _stage_c(agg_p, den_p,
                    bias.reshape(HEADS, OUT_DIM),
                    ln_gamma.reshape(HEADS, OUT_DIM),
                    ln_beta.reshape(HEADS, OUT_DIM))
    return out8[:, :N, :].transpose(1, 0, 2).reshape(N, HD)


# speedup vs baseline: 16.4600x; 1.0090x over previous
import functools

import jax
import jax.numpy as jnp
from jax import lax
from jax.experimental import pallas as pl
from jax.experimental.pallas import tpu as pltpu
from jax.experimental.pallas import tpu_sc as plsc

N = 10000
E = 160000
IN_DIM = 256
HEADS = 8
OUT_DIM = 64
HD = HEADS * OUT_DIM
NEG_SLOPE = 0.2
LN_EPS = 1e-5

N_PAD = 10240
NTILES = 32
ROWS_PT = N_PAD // 16
CB = 128
EPT = 5376
NCH = EPT // CB
NPAIR = NCH // 2
E_PAD = NTILES * EPT

RA = 1280
GROUPS = 8
ZROWS = 128


def _stage_a_body(x_ref, w_ref, asrc_w_ref, adst_w_ref, h_ref, asrc_ref, adst_ref):
    h = jnp.dot(x_ref[...], w_ref[...], preferred_element_type=jnp.float32)
    h_ref[...] = h
    h3 = h.reshape(RA, HEADS, OUT_DIM)
    a_s = jnp.sum(h3 * asrc_w_ref[...][None, :, :], axis=-1)
    a_d = jnp.sum(h3 * adst_w_ref[...][None, :, :], axis=-1)
    asrc_ref[...] = jnp.concatenate([a_s, a_s], axis=1)
    adst_ref[...] = jnp.concatenate([a_d, a_d], axis=1)


_stage_a = pl.pallas_call(
    _stage_a_body,
    grid=(N_PAD // RA,),
    in_specs=[
        pl.BlockSpec((RA, IN_DIM), lambda i: (i, 0)),
        pl.BlockSpec((IN_DIM, HD), lambda i: (0, 0)),
        pl.BlockSpec((HEADS, OUT_DIM), lambda i: (0, 0)),
        pl.BlockSpec((HEADS, OUT_DIM), lambda i: (0, 0)),
    ],
    out_specs=[
        pl.BlockSpec((RA, HD), lambda i: (i, 0)),
        pl.BlockSpec((RA, 16), lambda i: (i, 0)),
        pl.BlockSpec((RA, 16), lambda i: (i, 0)),
    ],
    out_shape=[
        jax.ShapeDtypeStruct((N_PAD, HD), jnp.float32),
        jax.ShapeDtypeStruct((N_PAD, 16), jnp.float32),
        jax.ShapeDtypeStruct((N_PAD, 16), jnp.float32),
    ],
)


def _sc_body(src_hbm, dst_hbm, asrc_hbm, adst_hbm, h_hbm,
             den_out, agg_out,
             srcF, dstF, dstC, gidxP, asr, adr, asrB, adrB,
             wbuf, wall, hrA, hrB, zb,
             den_sh, agg_sh,
             semL, semGA, semGB, semSA, semSB):
    cid = lax.axis_index("c")
    sid = lax.axis_index("s")
    tid = cid * 16 + sid
    row0 = sid * ROWS_PT

    z16 = jnp.zeros((16,), jnp.float32)
    m8 = lax.iota(jnp.int32, 16) < 8

    def zfill(r, carry):
        for k in range(4):
            zb[r, pl.ds(k * 16, 16)] = z16
        return carry

    lax.fori_loop(0, ZROWS, zfill, 0)

    for zi in range(ROWS_PT // ZROWS):
        pltpu.sync_copy(zb.at[pl.ds(0, ZROWS), pl.ds(0, 16)],
                        den_sh.at[pl.ds(row0 + zi * ZROWS, ZROWS)])
    plsc.subcore_barrier()

    dp1 = pltpu.async_copy(src_hbm.at[pl.ds(tid * EPT, 2 * CB)], srcF, semL)
    dp2 = pltpu.async_copy(dst_hbm.at[pl.ds(tid * EPT, 2 * CB)], dstF, semL)

    def pass1_pair(j, carry):
        dp1.wait()
        dp2.wait()
        for q in range(2):
            for k in range(CB // 16):
                dstC[q, pl.ds(k * 16, 16)] = dstF[pl.ds(q * CB + k * 16, 16)]
        gA1 = pltpu.async_copy(asrc_hbm.at[srcF.at[pl.ds(0, CB)]], asr, semGA)
        gA2 = pltpu.async_copy(adst_hbm.at[dstC.at[0]], adr, semGB)
        gB1 = pltpu.async_copy(asrc_hbm.at[srcF.at[pl.ds(CB, CB)]], asrB, semSA)
        gB2 = pltpu.async_copy(adst_hbm.at[dstC.at[1]], adrB, semSB)
        gA1.wait()
        gA2.wait()

        def edge2(f, c2):
            e0 = 2 * f
            s0 = asr[e0, :] + adr[e0, :]
            s0 = jnp.where(s0 >= 0.0, s0, s0 * NEG_SLOPE)
            w0 = jnp.exp(s0)
            wbuf[e0, :] = w0
            s1 = asr[e0 + 1, :] + adr[e0 + 1, :]
            s1 = jnp.where(s1 >= 0.0, s1, s1 * NEG_SLOPE)
            w1 = jnp.exp(s1)
            wbuf[e0 + 1, :] = w1
            wall[pl.ds((2 * j * CB + e0) * 8, 16)] = jnp.where(m8, w0, w1)
            return c2

        lax.fori_loop(0, CB // 2, edge2, 0)
        pltpu.sync_copy(wbuf, den_sh.at[dstC.at[0]], add=True)
        gB1.wait()
        gB2.wait()

        @pl.when(j + 1 < NPAIR)
        def _():
            nb = tid * EPT + (2 * j + 2) * CB
            pltpu.async_copy(src_hbm.at[pl.ds(nb, 2 * CB)], srcF, semL)
            pltpu.async_copy(dst_hbm.at[pl.ds(nb, 2 * CB)], dstF, semL)

        def edge2b(f, c2):
            e0 = 2 * f
            s0 = asrB[e0, :] + adrB[e0, :]
            s0 = jnp.where(s0 >= 0.0, s0, s0 * NEG_SLOPE)
            w0 = jnp.exp(s0)
            wbuf[e0, :] = w0
            s1 = asrB[e0 + 1, :] + adrB[e0 + 1, :]
            s1 = jnp.where(s1 >= 0.0, s1, s1 * NEG_SLOPE)
            w1 = jnp.exp(s1)
            wbuf[e0 + 1, :] = w1
            wall[pl.ds(((2 * j + 1) * CB + e0) * 8, 16)] = jnp.where(m8, w0, w1)
            return c2

        lax.fori_loop(0, CB // 2, edge2b, 0)
        pltpu.sync_copy(wbuf, den_sh.at[dstC.at[1]], add=True)
        return carry

    lax.fori_loop(0, NPAIR, pass1_pair, 0)
    plsc.subcore_barrier()
    pltpu.sync_copy(den_sh.at[pl.ds(row0, ROWS_PT)],
                    den_out.at[cid, pl.ds(row0, ROWS_PT)])

    for g in range(GROUPS):
        for zi in range(ROWS_PT // ZROWS):
            pltpu.sync_copy(zb, agg_sh.at[pl.ds(row0 + zi * ZROWS, ZROWS)])
        plsc.subcore_barrier()

        dL1 = pltpu.async_copy(src_hbm.at[pl.ds(tid * EPT, 2 * CB)], srcF, semL)
        dL2 = pltpu.async_copy(dst_hbm.at[pl.ds(tid * EPT, 2 * CB)], dstF, semL)

        def pair(j, carry, g=g):
            dL1.wait()
            dL2.wait()
            for k in range(2 * CB // 16):
                gidxP[pl.ds(k * 16, 16)] = srcF[pl.ds(k * 16, 16)] * GROUPS + g
            for r in range(2):
                for k in range(CB // 16):
                    dstC[r, pl.ds(k * 16, 16)] = dstF[pl.ds(r * CB + k * 16, 16)]
            dgA = pltpu.async_copy(h_hbm.at[gidxP.at[pl.ds(0, CB)]], hrA, semGA)
            dgB = pltpu.async_copy(h_hbm.at[gidxP.at[pl.ds(CB, CB)]], hrB, semGB)

            @pl.when(j + 1 < NPAIR)
            def _():
                nbase = tid * EPT + (2 * j + 2) * CB
                pltpu.async_copy(src_hbm.at[pl.ds(nbase, 2 * CB)], srcF, semL)
                pltpu.async_copy(dst_hbm.at[pl.ds(nbase, 2 * CB)], dstF, semL)

            dgA.wait()

            def scaleA(e, c2, g=g):
                w0 = wall[pl.ds((2 * j * CB + e) * 8, 16)][g]
                for k in range(4):
                    hrA[e, pl.ds(k * 16, 16)] = hrA[e, pl.ds(k * 16, 16)] * w0
                return c2

            lax.fori_loop(0, CB, scaleA, 0, unroll=4)
            dsA = pltpu.async_copy(hrA, agg_sh.at[dstC.at[0]], semSA, add=True)

            dgB.wait()

            def scaleB(e, c2, g=g):
                w0 = wall[pl.ds(((2 * j + 1) * CB + e) * 8, 16)][g]
                for k in range(4):
                    hrB[e, pl.ds(k * 16, 16)] = hrB[e, pl.ds(k * 16, 16)] * w0
                return c2

            lax.fori_loop(0, CB, scaleB, 0, unroll=4)
            dsB = pltpu.async_copy(hrB, agg_sh.at[dstC.at[1]], semSB, add=True)
            dsA.wait()
            dsB.wait()
            return carry

        lax.fori_loop(0, NPAIR, pair, 0)
        plsc.subcore_barrier()
        pltpu.sync_copy(agg_sh.at[pl.ds(row0, ROWS_PT)],
                        agg_out.at[cid, g, pl.ds(row0, ROWS_PT)])
        plsc.subcore_barrier()


_stage_b = pl.kernel(
    _sc_body,
    out_type=(
        jax.ShapeDtypeStruct((2, N_PAD, 16), jnp.float32),
        jax.ShapeDtypeStruct((2, HEADS, N_PAD, OUT_DIM), jnp.float32),
    ),
    mesh=plsc.VectorSubcoreMesh(core_axis_name="c", subcore_axis_name="s"),
    compiler_params=pltpu.CompilerParams(use_tc_tiling_on_sc=False),
    scratch_types=[
        pltpu.VMEM((2 * CB,), jnp.int32),
        pltpu.VMEM((2 * CB,), jnp.int32),
        pltpu.VMEM((2, CB), jnp.int32),
        pltpu.VMEM((2 * CB,), jnp.int32),
        pltpu.VMEM((CB, 16), jnp.float32),
        pltpu.VMEM((CB, 16), jnp.float32),
        pltpu.VMEM((CB, 16), jnp.float32),
        pltpu.VMEM((CB, 16), jnp.float32),
        pltpu.VMEM((CB, 16), jnp.float32),
        pltpu.VMEM((EPT * 8,), jnp.float32),
        pltpu.VMEM((CB, OUT_DIM), jnp.float32),
        pltpu.VMEM((CB, OUT_DIM), jnp.float32),
        pltpu.VMEM((ZROWS, OUT_DIM), jnp.float32),
        pltpu.VMEM_SHARED((N_PAD, 16), jnp.float32),
        pltpu.VMEM_SHARED((N_PAD, OUT_DIM), jnp.float32),
        pltpu.SemaphoreType.DMA,
        pltpu.SemaphoreType.DMA,
        pltpu.SemaphoreType.DMA,
        pltpu.SemaphoreType.DMA,
        pltpu.SemaphoreType.DMA,
    ],
)


def _stage_c_body(agg_ref, den_ref, bias_ref, gamma_ref, beta_ref, out_ref):
    s = agg_ref[0] + agg_ref[1]
    den16 = den_ref[0] + den_ref[1]
    eye = jnp.eye(HEADS, 16, dtype=jnp.float32)
    dent = lax.dot_general(eye, den16, (((1,), (1,)), ((), ())),
                           preferred_element_type=jnp.float32)
    y = s / dent[:, :, None] + bias_ref[...][:, None, :]
    t = jnp.sum(y, axis=0)
    mu = jnp.sum(t, axis=1) * (1.0 / HD)
    yc = y - mu[None, :, None]
    t2 = jnp.sum(yc * yc, axis=0)
    var = jnp.sum(t2, axis=1) * (1.0 / HD)
    inv = lax.rsqrt(var + LN_EPS)
    out_ref[...] = (yc * inv[None, :, None] * gamma_ref[...][:, None, :]
                    + beta_ref[...][:, None, :])


_stage_c = pl.pallas_call(
    _stage_c_body,
    grid=(N_PAD // RA,),
    in_specs=[
        pl.BlockSpec((2, HEADS, RA, OUT_DIM), lambda i: (0, 0, i, 0)),
        pl.BlockSpec((2, RA, 16), lambda i: (0, i, 0)),
        pl.BlockSpec((HEADS, OUT_DIM), lambda i: (0, 0)),
        pl.BlockSpec((HEADS, OUT_DIM), lambda i: (0, 0)),
        pl.BlockSpec((HEADS, OUT_DIM), lambda i: (0, 0)),
    ],
    out_specs=pl.BlockSpec((HEADS, RA, OUT_DIM), lambda i: (0, i, 0)),
    out_shape=jax.ShapeDtypeStruct((HEADS, N_PAD, OUT_DIM), jnp.float32),
)


def kernel(x, edge_index, W, att_src, att_dst, bias, ln_gamma, ln_beta):
    x_pad = jnp.zeros((N_PAD, IN_DIM), jnp.float32).at[:N].set(x)
    h, asrc, adst = _stage_a(x_pad, W, att_src, att_dst)

    src = edge_index[0].astype(jnp.int32)
    dst = edge_index[1].astype(jnp.int32)
    loop = jnp.arange(N, dtype=jnp.int32)
    pad = jnp.full((E_PAD - E - N,), N, dtype=jnp.int32)
    src_full = jnp.concatenate([src, loop, pad])
    dst_full = jnp.concatenate([dst, loop, pad])

    h2d = h.reshape(N_PAD * GROUPS, OUT_DIM)
    den_p, agg_p = _stage_b(src_full, dst_full, asrc, adst, h2d)

    out8 = _stage_c(agg_p, den_p,
                    bias.reshape(HEADS, OUT_DIM),
                    ln_gamma.reshape(HEADS, OUT_DIM),
                    ln_beta.reshape(HEADS, OUT_DIM))
    return out8[:, :N, :].transpose(1, 0, 2).reshape(N, HD)
